# baseline (speedup 1.0000x reference)
import functools

import jax
import jax.numpy as jnp
from jax import lax
from jax.experimental import pallas as pl
from jax.experimental.pallas import tpu as pltpu
from jax.experimental.pallas import tpu_sc as plsc

BATCH = 16384
EMBED_DIM = 32
ROWS_PER_128 = 128 // EMBED_DIM
NUM_CORES = 2
NUM_SUBCORES = 16
NUM_WORKERS = NUM_CORES * NUM_SUBCORES
B_PER_W = BATCH // NUM_WORKERS
LANES = 16


def _sc_gather128(table128, team_ids):
    mesh = plsc.VectorSubcoreMesh(core_axis_name="c", subcore_axis_name="s")

    @functools.partial(
        pl.kernel,
        mesh=mesh,
        out_type=jax.ShapeDtypeStruct((BATCH, 128), jnp.float32),
        scratch_types=[
            pltpu.VMEM((B_PER_W,), jnp.int32),
            pltpu.VMEM((B_PER_W,), jnp.int32),
            pltpu.VMEM((B_PER_W, 128), jnp.float32),
            pltpu.SemaphoreType.DMA,
        ],
    )
    def k(table_hbm, idx_hbm, out_hbm, idx_v, idq_v, rows_v, sem):
        wid = lax.axis_index("s") * NUM_CORES + lax.axis_index("c")
        base = wid * B_PER_W
        pltpu.sync_copy(idx_hbm.at[pl.ds(base, B_PER_W)], idx_v)
        for i in range(B_PER_W // LANES):
            sl = pl.ds(i * LANES, LANES)
            idq_v[sl] = lax.shift_right_logical(idx_v[sl], 2)
        pltpu.async_copy(table_hbm.at[idq_v], rows_v, sem).wait()
        pltpu.sync_copy(rows_v, out_hbm.at[pl.ds(base, B_PER_W)])

    return k(table128, team_ids)


ROW_BLOCK = 2048


def _extract_dense_body(g_ref, ids_ref, stats_ref, w_ref, b_ref, out_ref):
    off = ids_ref[...] % ROWS_PER_128
    emb = jnp.zeros((ROW_BLOCK, EMBED_DIM), jnp.float32)
    for k in range(ROWS_PER_128):
        part = g_ref[:, k * EMBED_DIM:(k + 1) * EMBED_DIM]
        emb = jnp.where(off == k, part, emb)
    out_ref[...] = emb + lax.dot_general(
        stats_ref[...], w_ref[...],
        dimension_numbers=(((1,), (1,)), ((), ())),
        preferred_element_type=jnp.float32,
    ) + b_ref[...]


def _extract_dense(gathered128, team_ids, team_stats, W, b):
    nblk = BATCH // ROW_BLOCK
    return pl.pallas_call(
        _extract_dense_body,
        grid=(nblk,),
        in_specs=[
            pl.BlockSpec((ROW_BLOCK, 128), lambda i: (i, 0)),
            pl.BlockSpec((ROW_BLOCK, 1), lambda i: (i, 0)),
            pl.BlockSpec((ROW_BLOCK, 10), lambda i: (i, 0)),
            pl.BlockSpec((EMBED_DIM, 10), lambda i: (0, 0)),
            pl.BlockSpec((1, EMBED_DIM), lambda i: (0, 0)),
        ],
        out_specs=pl.BlockSpec((ROW_BLOCK, EMBED_DIM), lambda i: (i, 0)),
        out_shape=jax.ShapeDtypeStruct((BATCH, EMBED_DIM), jnp.float32),
    )(gathered128, team_ids.reshape(BATCH, 1), team_stats, W,
      b.reshape(1, EMBED_DIM))


def kernel(team_ids, team_stats, emb_table, W, b):
    ids32 = team_ids.astype(jnp.int32)
    table128 = emb_table.reshape(emb_table.shape[0] // ROWS_PER_128, 128)
    gathered128 = _sc_gather128(table128, ids32)
    return _extract_dense(gathered128, ids32, team_stats, W, b)
